# baseline (speedup 1.0000x reference)
import functools

import jax
import jax.numpy as jnp
from jax import lax
from jax.experimental import pallas as pl
from jax.experimental.pallas import tpu as pltpu
from jax.experimental.pallas import tpu_sc as plsc

N = 5000
NPAD = 5120
C = 3
M = 100
MPAD = 104
SCORE_THR = 0.05
NMS_THR = 0.5
L = 16
TILES = 16
PB = NPAD // TILES
PV = PB // L
RW = 8
SLOT = TILES * RW
NEG = -1e30
BIGF = 1e9

_DNUMS = lax.GatherDimensionNumbers(
    offset_dims=(), collapsed_slice_dims=(0,), start_index_map=(0,))


def _iota16():
    return lax.broadcasted_iota(jnp.int32, (L,), 0)


def _perm(v, idx):
    return lax.gather(v, idx.reshape(L, 1), _DNUMS, (1,),
                      mode=lax.GatherScatterMode.PROMISE_IN_BOUNDS)


def _allreduce(v, op, lanes):
    for s in (1, 2, 4, 8):
        v = op(v, _perm(v, lanes ^ s))
    return v


def _sc_body(bbp_hbm, nmsT_hbm, scT_hbm, out_hbm, xcf_r, ycf_r, wf_r, hf_r,
             x1_r, y1_r, x2_r, y2_r, ar_r, pos_r, msc_r, rec_r,
             cand_sh, cand_l, idx_r, patch_r, vmask_r, rows_r, outb_r, sem,
             sem2):
    cid = lax.axis_index("c")
    sid = lax.axis_index("s")

    def core_run(classes, out_off):
        CL = len(classes)
        base = sid * PB
        lanes = _iota16()

        pltpu.sync_copy(nmsT_hbm.at[pl.ds(0 * NPAD, NPAD)], xcf_r)
        pltpu.sync_copy(nmsT_hbm.at[pl.ds(1 * NPAD, NPAD)], ycf_r)
        pltpu.sync_copy(nmsT_hbm.at[pl.ds(2 * NPAD, NPAD)], wf_r)
        pltpu.sync_copy(nmsT_hbm.at[pl.ds(3 * NPAD, NPAD)], hf_r)
        for p, c in enumerate(classes):
            pltpu.sync_copy(scT_hbm.at[pl.ds(c * NPAD + base, PB)],
                            msc_r.at[pl.ds(p * PB, PB)])
        for j in range(PV):
            sl = pl.ds(j * L, L)
            fsl = pl.ds(base + j * L, L)
            xc = xcf_r[fsl]
            w = wf_r[fsl]
            hw = w / 2.0
            x1 = xc - hw
            x2 = xc + hw
            x1_r[sl] = x1
            x2_r[sl] = x2
            yc = ycf_r[fsl]
            h = hf_r[fsl]
            hh = h / 2.0
            y1 = yc - hh
            y2 = yc + hh
            y1_r[sl] = y1
            y2_r[sl] = y2
            ar_r[sl] = (x2 - x1) * (y2 - y1)
            pos_r[sl] = (base + j * L + lanes).astype(jnp.float32)
            for p in range(CL):
                v = msc_r[pl.ds(p * PB + j * L, L)]
                msc_r[pl.ds(p * PB + j * L, L)] = jnp.where(
                    v > SCORE_THR, v, NEG)

        def pack_rec(p, bv, bp, rec):
            m = _allreduce(bv, jnp.maximum, lanes)
            gposf = _allreduce(jnp.where(bv == m, bp, BIGF), jnp.minimum,
                               lanes)
            return jnp.where(lanes == 2 * p, m,
                             jnp.where(lanes == 2 * p + 1, gposf, rec))

        rec = jnp.zeros((L,), jnp.float32)
        for p in range(CL):
            def amax_body(j, st):
                bv, bp = st
                sl = pl.ds(p * PB + j * L, L)
                ms = msc_r[sl]
                posf = (base + j * L + lanes).astype(jnp.float32)
                upd = ms > bv
                return (jnp.where(upd, ms, bv), jnp.where(upd, posf, bp))

            bv, bp = lax.fori_loop(
                0, PV, amax_body,
                (jnp.full((L,), NEG, jnp.float32),
                 jnp.full((L,), BIGF, jnp.float32)))
            rec = pack_rec(p, bv, bp, rec)
        rec_r[...] = rec
        pltpu.sync_copy(rec_r.at[pl.ds(0, RW)], cand_sh.at[pl.ds(sid * RW, RW)])
        plsc.subcore_barrier()

        def round_body(i, carry):
            slot = (i % 2) * SLOT
            nslot = ((i + 1) % 2) * SLOT
            pltpu.sync_copy(cand_sh.at[pl.ds(slot, SLOT)], cand_l)
            winners = []
            for p in range(CL):
                mcol = plsc.load_gather(cand_l, [lanes * RW + 2 * p])
                gcol = plsc.load_gather(cand_l, [lanes * RW + 2 * p + 1])
                gmax = _allreduce(mcol, jnp.maximum, lanes)
                validb = gmax > -1e29
                wg = _allreduce(jnp.where(mcol == gmax, gcol, BIGF),
                                jnp.minimum, lanes)
                wg_i = jnp.where(validb, wg.astype(jnp.int32), 0)
                wxc = plsc.load_gather(xcf_r, [wg_i])
                wyc = plsc.load_gather(ycf_r, [wg_i])
                ww = plsc.load_gather(wf_r, [wg_i])
                wh = plsc.load_gather(hf_r, [wg_i])
                whw = ww / 2.0
                whh = wh / 2.0
                wx1 = wxc - whw
                wx2 = wxc + whw
                wy1 = wyc - whh
                wy2 = wyc + whh
                warea = (wx2 - wx1) * (wy2 - wy1)
                wx1 = jnp.where(validb, wx1, BIGF)
                wx2 = jnp.where(validb, wx2, -BIGF)
                lpos = jnp.clip(wg_i - base, 0, PB - 1) + p * PB
                plsc.store_scatter(
                    msc_r, [lpos], jnp.full((L,), NEG, jnp.float32),
                    mask=(lanes == 0) & (wg_i // PB == sid))
                winners.append((gmax, validb, wg, wg_i, wx1, wx2, wy1, wy2,
                                warea))

            bvs = [jnp.full((L,), NEG, jnp.float32) for _ in range(CL)]
            bps = [jnp.full((L,), BIGF, jnp.float32) for _ in range(CL)]
            for j in range(PV):
                sl = pl.ds(j * L, L)
                xx1 = x1_r[sl]
                yy1 = y1_r[sl]
                xx2 = x2_r[sl]
                yy2 = y2_r[sl]
                aj = ar_r[sl]
                gpos = pos_r[sl]
                for p in range(CL):
                    (gmax, validb, wg, wg_i, wx1, wx2, wy1, wy2,
                     warea) = winners[p]
                    asl = pl.ds(p * PB + j * L, L)
                    iw = jnp.maximum(
                        jnp.minimum(wx2, xx2) - jnp.maximum(wx1, xx1), 0.0)
                    ih = jnp.maximum(
                        jnp.minimum(wy2, yy2) - jnp.maximum(wy1, yy1), 0.0)
                    inter = iw * ih
                    union = aj + warea - inter
                    iou = inter / jnp.maximum(union, 1e-8)
                    supp = iou > NMS_THR
                    ms = jnp.where(supp, NEG, msc_r[asl])
                    msc_r[asl] = ms
                    upd = ms > bvs[p]
                    bvs[p] = jnp.where(upd, ms, bvs[p])
                    bps[p] = jnp.where(upd, gpos, bps[p])

            rec = jnp.zeros((L,), jnp.float32)
            for p in range(CL):
                rec = pack_rec(p, bvs[p], bps[p], rec)
            rec_r[...] = rec
            cp = pltpu.make_async_copy(
                rec_r.at[pl.ds(0, RW)],
                cand_sh.at[pl.ds(nslot + sid * RW, RW)], sem2)
            cp.start()

            @pl.when(sid == 0)
            def _():
                for p, c in enumerate(classes):
                    gmax, validb, wg, wg_i = winners[p][:4]
                    scorep = jnp.where(validb, gmax, 0.0)
                    lab = jnp.where(validb, jnp.float32(c), 0.0)
                    patch = jnp.where(
                        lanes == 7, scorep,
                        jnp.where(lanes == 8, lab, jnp.float32(0.0)))
                    patch_r[pl.ds(i * L + p * M * L, L)] = patch
                    vmask_r[pl.ds(i * L + p * M * L, L)] = jnp.where(
                        validb, 1.0, 0.0)
                    plsc.store_scatter(
                        idx_r, [jnp.broadcast_to(p * MPAD + i, (L,))],
                        wg_i, mask=lanes == 0)

            cp.wait()
            plsc.subcore_barrier()
            return carry

        lax.fori_loop(0, M, round_body, 0)

        @pl.when(sid == 0)
        def _():
            for p in range(CL):
                pltpu.async_copy(
                    bbp_hbm.at[idx_r.at[pl.ds(p * MPAD, M)]], rows_r,
                    sem).wait()

                def emit_body(r, carry3):
                    row16 = plsc.load_gather(
                        rows_r, [jnp.broadcast_to(r, (L,)), lanes])
                    psl = pl.ds(r * L + p * M * L, L)
                    outb_r[psl] = row16 * vmask_r[psl] + patch_r[psl]
                    return carry3

                lax.fori_loop(0, M, emit_body, 0)
            pltpu.sync_copy(outb_r.at[pl.ds(0, CL * M * L)],
                            out_hbm.at[pl.ds(out_off * L, CL * M * L)])

    @pl.when(cid == 0)
    def _():
        core_run((0, 1), 0)

    @pl.when(cid == 1)
    def _():
        core_run((2,), 2 * M)


@jax.jit
def kernel(mlvl_bboxes, mlvl_bboxes_for_nms, mlvl_scores):
    scT = jnp.zeros((C, NPAD), jnp.float32).at[:, :N].set(
        mlvl_scores[:, :C].T).reshape(C * NPAD)
    nmsT = jnp.zeros((4, NPAD), jnp.float32).at[:, :N].set(
        mlvl_bboxes_for_nms[:, :4].T).reshape(4 * NPAD)
    bbp = jnp.zeros((NPAD, 128), jnp.float32).at[:N, :7].set(mlvl_bboxes)
    mesh = plsc.VectorSubcoreMesh(core_axis_name="c", subcore_axis_name="s")
    out = pl.kernel(
        _sc_body,
        out_type=jax.ShapeDtypeStruct((C * M * L,), jnp.float32),
        mesh=mesh,
        compiler_params=pltpu.CompilerParams(needs_layout_passes=False),
        scratch_types=[
            pltpu.VMEM((NPAD,), jnp.float32),
            pltpu.VMEM((NPAD,), jnp.float32),
            pltpu.VMEM((NPAD,), jnp.float32),
            pltpu.VMEM((NPAD,), jnp.float32),
            pltpu.VMEM((PB,), jnp.float32),
            pltpu.VMEM((PB,), jnp.float32),
            pltpu.VMEM((PB,), jnp.float32),
            pltpu.VMEM((PB,), jnp.float32),
            pltpu.VMEM((PB,), jnp.float32),
            pltpu.VMEM((PB,), jnp.float32),
            pltpu.VMEM((C * PB,), jnp.float32),
            pltpu.VMEM((L,), jnp.float32),
            pltpu.VMEM_SHARED((2 * SLOT,), jnp.float32),
            pltpu.VMEM((SLOT,), jnp.float32),
            pltpu.VMEM((C * MPAD,), jnp.int32),
            pltpu.VMEM((C * M * L,), jnp.float32),
            pltpu.VMEM((C * M * L,), jnp.float32),
            pltpu.VMEM((M, 128), jnp.float32),
            pltpu.VMEM((C * M * L,), jnp.float32),
            pltpu.SemaphoreType.DMA,
            pltpu.SemaphoreType.DMA,
        ],
    )(bbp, nmsT, scT)
    return out.reshape(C * M, L)[:, :9]


# speedup vs baseline: 1.0850x; 1.0850x over previous
import functools

import jax
import jax.numpy as jnp
from jax import lax
from jax.experimental import pallas as pl
from jax.experimental.pallas import tpu as pltpu
from jax.experimental.pallas import tpu_sc as plsc

N = 5000
NPAD = 5120
C = 3
M = 100
MPAD = 104
SCORE_THR = 0.05
NMS_THR = 0.5
L = 16
TILES = 16
PB = NPAD // TILES
PV = PB // L
RW = 8
SLOT = TILES * RW
NEG = -1e30
BIGF = 1e9

_DNUMS = lax.GatherDimensionNumbers(
    offset_dims=(), collapsed_slice_dims=(0,), start_index_map=(0,))


def _iota16():
    return lax.broadcasted_iota(jnp.int32, (L,), 0)


def _perm(v, idx):
    return lax.gather(v, idx.reshape(L, 1), _DNUMS, (1,),
                      mode=lax.GatherScatterMode.PROMISE_IN_BOUNDS)


def _allreduce(v, op, lanes):
    for s in (1, 2, 4, 8):
        v = op(v, _perm(v, lanes ^ s))
    return v


def _sc_body(bbp_hbm, nmsT_hbm, scT_hbm, out_hbm, xcf_r, ycf_r, wf_r, hf_r,
             x1_r, y1_r, x2_r, y2_r, ar_r, pos_r, msc_r, rec_r,
             cand_sh, cand_l, idx_r, patch_r, vmask_r, rows_r, outb_r, sem,
             sem2, sem3):
    cid = lax.axis_index("c")
    sid = lax.axis_index("s")

    def core_run(classes, out_off):
        CL = len(classes)
        base = sid * PB
        lanes = _iota16()

        pltpu.sync_copy(nmsT_hbm.at[pl.ds(0 * NPAD, NPAD)], xcf_r)
        pltpu.sync_copy(nmsT_hbm.at[pl.ds(1 * NPAD, NPAD)], ycf_r)
        pltpu.sync_copy(nmsT_hbm.at[pl.ds(2 * NPAD, NPAD)], wf_r)
        pltpu.sync_copy(nmsT_hbm.at[pl.ds(3 * NPAD, NPAD)], hf_r)
        for p, c in enumerate(classes):
            pltpu.sync_copy(scT_hbm.at[pl.ds(c * NPAD + base, PB)],
                            msc_r.at[pl.ds(p * PB, PB)])
        for j in range(PV):
            sl = pl.ds(j * L, L)
            fsl = pl.ds(base + j * L, L)
            xc = xcf_r[fsl]
            w = wf_r[fsl]
            hw = w / 2.0
            x1 = xc - hw
            x2 = xc + hw
            x1_r[sl] = x1
            x2_r[sl] = x2
            yc = ycf_r[fsl]
            h = hf_r[fsl]
            hh = h / 2.0
            y1 = yc - hh
            y2 = yc + hh
            y1_r[sl] = y1
            y2_r[sl] = y2
            ar_r[sl] = (x2 - x1) * (y2 - y1)
            pos_r[sl] = (base + j * L + lanes).astype(jnp.float32)
            for p in range(CL):
                v = msc_r[pl.ds(p * PB + j * L, L)]
                msc_r[pl.ds(p * PB + j * L, L)] = jnp.where(
                    v > SCORE_THR, v, NEG)

        def pack_rec(p, bv, bp, rec):
            m = _allreduce(bv, jnp.maximum, lanes)
            gposf = _allreduce(jnp.where(bv == m, bp, BIGF), jnp.minimum,
                               lanes)
            return jnp.where(lanes == 2 * p, m,
                             jnp.where(lanes == 2 * p + 1, gposf, rec))

        rec = jnp.zeros((L,), jnp.float32)
        for p in range(CL):
            def amax_body(j, st):
                bv, bp = st
                sl = pl.ds(p * PB + j * L, L)
                ms = msc_r[sl]
                posf = (base + j * L + lanes).astype(jnp.float32)
                upd = ms > bv
                return (jnp.where(upd, ms, bv), jnp.where(upd, posf, bp))

            bv, bp = lax.fori_loop(
                0, PV, amax_body,
                (jnp.full((L,), NEG, jnp.float32),
                 jnp.full((L,), BIGF, jnp.float32)))
            rec = pack_rec(p, bv, bp, rec)
        rec_r[...] = rec
        pltpu.sync_copy(rec_r.at[pl.ds(0, RW)], cand_sh.at[pl.ds(sid * RW, RW)])
        plsc.subcore_barrier()

        def book(i, bk):
            for p in range(CL):
                patch, vmaskf, wg_i = bk[3 * p:3 * p + 3]
                patch_r[pl.ds(i * L + p * M * L, L)] = patch
                vmask_r[pl.ds(i * L + p * M * L, L)] = vmaskf
                plsc.store_scatter(
                    idx_r, [jnp.broadcast_to(p * MPAD + i, (L,))],
                    wg_i, mask=lanes == 0)

        def round_body(i, bk):
            slot = (i % 2) * SLOT
            nslot = ((i + 1) % 2) * SLOT
            cp0 = pltpu.make_async_copy(
                cand_sh.at[pl.ds(slot, SLOT)], cand_l, sem3)
            cp0.start()

            @pl.when((sid == 0) & (i > 0))
            def _():
                book(i - 1, bk)

            cp0.wait()
            winners = []
            for p in range(CL):
                mcol = plsc.load_gather(cand_l, [lanes * RW + 2 * p])
                gcol = plsc.load_gather(cand_l, [lanes * RW + 2 * p + 1])
                gmax = _allreduce(mcol, jnp.maximum, lanes)
                validb = gmax > -1e29
                wg = _allreduce(jnp.where(mcol == gmax, gcol, BIGF),
                                jnp.minimum, lanes)
                wg_i = jnp.where(validb, wg.astype(jnp.int32), 0)
                wxc = plsc.load_gather(xcf_r, [wg_i])
                wyc = plsc.load_gather(ycf_r, [wg_i])
                ww = plsc.load_gather(wf_r, [wg_i])
                wh = plsc.load_gather(hf_r, [wg_i])
                whw = ww / 2.0
                whh = wh / 2.0
                wx1 = wxc - whw
                wx2 = wxc + whw
                wy1 = wyc - whh
                wy2 = wyc + whh
                warea = (wx2 - wx1) * (wy2 - wy1)
                wx1 = jnp.where(validb, wx1, BIGF)
                wx2 = jnp.where(validb, wx2, -BIGF)
                winners.append((gmax, validb, wg, wg_i, wx1, wx2, wy1, wy2,
                                warea))

            bvs = [jnp.full((L,), NEG, jnp.float32) for _ in range(CL)]
            bps = [jnp.full((L,), BIGF, jnp.float32) for _ in range(CL)]
            for j in range(PV):
                sl = pl.ds(j * L, L)
                xx1 = x1_r[sl]
                yy1 = y1_r[sl]
                xx2 = x2_r[sl]
                yy2 = y2_r[sl]
                aj = ar_r[sl]
                gpos = pos_r[sl]
                for p in range(CL):
                    (gmax, validb, wg, wg_i, wx1, wx2, wy1, wy2,
                     warea) = winners[p]
                    asl = pl.ds(p * PB + j * L, L)
                    iw = jnp.maximum(
                        jnp.minimum(wx2, xx2) - jnp.maximum(wx1, xx1), 0.0)
                    ih = jnp.maximum(
                        jnp.minimum(wy2, yy2) - jnp.maximum(wy1, yy1), 0.0)
                    inter = iw * ih
                    union = aj + warea - inter
                    iou = inter / jnp.maximum(union, 1e-8)
                    supp = (iou > NMS_THR) | (gpos == wg)
                    ms = jnp.where(supp, NEG, msc_r[asl])
                    msc_r[asl] = ms
                    upd = ms > bvs[p]
                    bvs[p] = jnp.where(upd, ms, bvs[p])
                    bps[p] = jnp.where(upd, gpos, bps[p])

            rec = jnp.zeros((L,), jnp.float32)
            for p in range(CL):
                rec = pack_rec(p, bvs[p], bps[p], rec)
            rec_r[...] = rec
            cp = pltpu.make_async_copy(
                rec_r.at[pl.ds(0, RW)],
                cand_sh.at[pl.ds(nslot + sid * RW, RW)], sem2)
            cp.start()

            nbk = []
            for p, c in enumerate(classes):
                gmax, validb, wg, wg_i = winners[p][:4]
                scorep = jnp.where(validb, gmax, 0.0)
                lab = jnp.where(validb, jnp.float32(c), 0.0)
                patch = jnp.where(
                    lanes == 7, scorep,
                    jnp.where(lanes == 8, lab, jnp.float32(0.0)))
                vmaskf = jnp.where(validb, 1.0, 0.0)
                nbk.extend([patch, vmaskf, wg_i])

            cp.wait()
            plsc.subcore_barrier()
            return tuple(nbk)

        zf = jnp.zeros((L,), jnp.float32)
        zi = jnp.zeros((L,), jnp.int32)
        bk0 = tuple([zf, zf, zi] * CL)
        bk_fin = lax.fori_loop(0, M, round_body, bk0)

        @pl.when(sid == 0)
        def _():
            book(M - 1, bk_fin)

        @pl.when(sid == 0)
        def _():
            for p in range(CL):
                pltpu.async_copy(
                    bbp_hbm.at[idx_r.at[pl.ds(p * MPAD, M)]], rows_r,
                    sem).wait()

                def emit_body(r, carry3):
                    row16 = plsc.load_gather(
                        rows_r, [jnp.broadcast_to(r, (L,)), lanes])
                    psl = pl.ds(r * L + p * M * L, L)
                    outb_r[psl] = row16 * vmask_r[psl] + patch_r[psl]
                    return carry3

                lax.fori_loop(0, M, emit_body, 0)
            pltpu.sync_copy(outb_r.at[pl.ds(0, CL * M * L)],
                            out_hbm.at[pl.ds(out_off * L, CL * M * L)])

    @pl.when(cid == 0)
    def _():
        core_run((0, 1), 0)

    @pl.when(cid == 1)
    def _():
        core_run((2,), 2 * M)


@jax.jit
def kernel(mlvl_bboxes, mlvl_bboxes_for_nms, mlvl_scores):
    scT = jnp.zeros((C, NPAD), jnp.float32).at[:, :N].set(
        mlvl_scores[:, :C].T).reshape(C * NPAD)
    nmsT = jnp.zeros((4, NPAD), jnp.float32).at[:, :N].set(
        mlvl_bboxes_for_nms[:, :4].T).reshape(4 * NPAD)
    bbp = jnp.zeros((NPAD, 128), jnp.float32).at[:N, :7].set(mlvl_bboxes)
    mesh = plsc.VectorSubcoreMesh(core_axis_name="c", subcore_axis_name="s")
    out = pl.kernel(
        _sc_body,
        out_type=jax.ShapeDtypeStruct((C * M * L,), jnp.float32),
        mesh=mesh,
        compiler_params=pltpu.CompilerParams(needs_layout_passes=False),
        scratch_types=[
            pltpu.VMEM((NPAD,), jnp.float32),
            pltpu.VMEM((NPAD,), jnp.float32),
            pltpu.VMEM((NPAD,), jnp.float32),
            pltpu.VMEM((NPAD,), jnp.float32),
            pltpu.VMEM((PB,), jnp.float32),
            pltpu.VMEM((PB,), jnp.float32),
            pltpu.VMEM((PB,), jnp.float32),
            pltpu.VMEM((PB,), jnp.float32),
            pltpu.VMEM((PB,), jnp.float32),
            pltpu.VMEM((PB,), jnp.float32),
            pltpu.VMEM((C * PB,), jnp.float32),
            pltpu.VMEM((L,), jnp.float32),
            pltpu.VMEM_SHARED((2 * SLOT,), jnp.float32),
            pltpu.VMEM((SLOT,), jnp.float32),
            pltpu.VMEM((C * MPAD,), jnp.int32),
            pltpu.VMEM((C * M * L,), jnp.float32),
            pltpu.VMEM((C * M * L,), jnp.float32),
            pltpu.VMEM((M, 128), jnp.float32),
            pltpu.VMEM((C * M * L,), jnp.float32),
            pltpu.SemaphoreType.DMA,
            pltpu.SemaphoreType.DMA,
            pltpu.SemaphoreType.DMA,
        ],
    )(bbp, nmsT, scT)
    return out.reshape(C * M, L)[:, :9]


# speedup vs baseline: 1.1148x; 1.0275x over previous
import jax
import jax.numpy as jnp
from jax import lax
from jax.experimental import pallas as pl
from jax.experimental.pallas import tpu as pltpu
from jax.experimental.pallas import tpu_sc as plsc

N = 5000
NPAD = 5120
C = 3
M = 100
MPAD = 104
SCORE_THR = 0.05
NMS_THR = 0.5
L = 16
TILES = 16
PB = NPAD // TILES
PV = PB // L
RW = 8
SLOT = TILES * RW
NEG = -1e30
BIGF = 1e9

_DNUMS = lax.GatherDimensionNumbers(
    offset_dims=(), collapsed_slice_dims=(0,), start_index_map=(0,))


def _iota16():
    return lax.broadcasted_iota(jnp.int32, (L,), 0)


def _perm(v, idx):
    return lax.gather(v, idx.reshape(L, 1), _DNUMS, (1,),
                      mode=lax.GatherScatterMode.PROMISE_IN_BOUNDS)


def _allreduce(v, op, lanes):
    for s in (1, 2, 4, 8):
        v = op(v, _perm(v, lanes ^ s))
    return v


def _sc_body(bbp_hbm, nmsT_hbm, scT_hbm, out_hbm, xcf_r, ycf_r, wf_r, hf_r,
             x1_r, y1_r, x2_r, y2_r, ar_r, pos_r, msc_r, rec_r,
             cand_sh, cand_l, idx_r, patch_r, vmask_r, rows_r, outb_r, sem,
             sem2):
    cid = lax.axis_index("c")
    sid = lax.axis_index("s")

    def core_run(classes, out_off):
        CL = len(classes)
        base = sid * PB
        lanes = _iota16()

        pltpu.sync_copy(nmsT_hbm.at[pl.ds(0 * NPAD, NPAD)], xcf_r)
        pltpu.sync_copy(nmsT_hbm.at[pl.ds(1 * NPAD, NPAD)], ycf_r)
        pltpu.sync_copy(nmsT_hbm.at[pl.ds(2 * NPAD, NPAD)], wf_r)
        pltpu.sync_copy(nmsT_hbm.at[pl.ds(3 * NPAD, NPAD)], hf_r)
        for p, c in enumerate(classes):
            pltpu.sync_copy(scT_hbm.at[pl.ds(c * NPAD + base, PB)],
                            msc_r.at[pl.ds(p * PB, PB)])
        for j in range(PV):
            sl = pl.ds(j * L, L)
            fsl = pl.ds(base + j * L, L)
            xc = xcf_r[fsl]
            w = wf_r[fsl]
            hw = w / 2.0
            x1 = xc - hw
            x2 = xc + hw
            x1_r[sl] = x1
            x2_r[sl] = x2
            yc = ycf_r[fsl]
            h = hf_r[fsl]
            hh = h / 2.0
            y1 = yc - hh
            y2 = yc + hh
            y1_r[sl] = y1
            y2_r[sl] = y2
            ar_r[sl] = (x2 - x1) * (y2 - y1)
            pos_r[sl] = (base + j * L + lanes).astype(jnp.float32)
            for p in range(CL):
                v = msc_r[pl.ds(p * PB + j * L, L)]
                msc_r[pl.ds(p * PB + j * L, L)] = jnp.where(
                    v > SCORE_THR, v, NEG)

        def pack_rec(p, bv, bp, rec):
            m = _allreduce(bv, jnp.maximum, lanes)
            gposf = _allreduce(jnp.where(bv == m, bp, BIGF), jnp.minimum,
                               lanes)
            return jnp.where(lanes == 2 * p, m,
                             jnp.where(lanes == 2 * p + 1, gposf, rec))

        rec = jnp.zeros((L,), jnp.float32)
        for p in range(CL):
            def amax_body(j, st):
                bv, bp = st
                sl = pl.ds(p * PB + j * L, L)
                ms = msc_r[sl]
                posf = (base + j * L + lanes).astype(jnp.float32)
                upd = ms > bv
                return (jnp.where(upd, ms, bv), jnp.where(upd, posf, bp))

            bv, bp = lax.fori_loop(
                0, PV, amax_body,
                (jnp.full((L,), NEG, jnp.float32),
                 jnp.full((L,), BIGF, jnp.float32)))
            rec = pack_rec(p, bv, bp, rec)
        rec_r[...] = rec
        pltpu.sync_copy(rec_r.at[pl.ds(0, RW)], cand_sh.at[pl.ds(sid * RW, RW)])
        plsc.subcore_barrier()

        def round_body(i, carry):
            slot = (i % 2) * SLOT
            nslot = ((i + 1) % 2) * SLOT
            pltpu.sync_copy(cand_sh.at[pl.ds(slot, SLOT)], cand_l)
            winners = []
            for p in range(CL):
                mcol = plsc.load_gather(cand_l, [lanes * RW + 2 * p])
                gcol = plsc.load_gather(cand_l, [lanes * RW + 2 * p + 1])
                gmax = _allreduce(mcol, jnp.maximum, lanes)
                validb = gmax > -1e29
                wg = _allreduce(jnp.where(mcol == gmax, gcol, BIGF),
                                jnp.minimum, lanes)
                wg_i = jnp.where(validb, wg.astype(jnp.int32), 0)
                wxc = plsc.load_gather(xcf_r, [wg_i])
                wyc = plsc.load_gather(ycf_r, [wg_i])
                ww = plsc.load_gather(wf_r, [wg_i])
                wh = plsc.load_gather(hf_r, [wg_i])
                whw = ww / 2.0
                whh = wh / 2.0
                wx1 = wxc - whw
                wx2 = wxc + whw
                wy1 = wyc - whh
                wy2 = wyc + whh
                warea = (wx2 - wx1) * (wy2 - wy1)
                wx1 = jnp.where(validb, wx1, BIGF)
                wx2 = jnp.where(validb, wx2, -BIGF)
                winners.append((gmax, validb, wg, wg_i, wx1, wx2, wy1, wy2,
                                warea))

            bvs = [jnp.full((L,), NEG, jnp.float32) for _ in range(CL)]
            bps = [jnp.full((L,), BIGF, jnp.float32) for _ in range(CL)]
            for j in range(PV):
                sl = pl.ds(j * L, L)
                xx1 = x1_r[sl]
                yy1 = y1_r[sl]
                xx2 = x2_r[sl]
                yy2 = y2_r[sl]
                aj = ar_r[sl]
                gpos = pos_r[sl]
                for p in range(CL):
                    (gmax, validb, wg, wg_i, wx1, wx2, wy1, wy2,
                     warea) = winners[p]
                    asl = pl.ds(p * PB + j * L, L)
                    iw = jnp.maximum(
                        jnp.minimum(wx2, xx2) - jnp.maximum(wx1, xx1), 0.0)
                    ih = jnp.maximum(
                        jnp.minimum(wy2, yy2) - jnp.maximum(wy1, yy1), 0.0)
                    inter = iw * ih
                    union = aj + warea - inter
                    iou = inter / jnp.maximum(union, 1e-8)
                    supp = (iou > NMS_THR) | (gpos == wg)
                    ms = jnp.where(supp, NEG, msc_r[asl])
                    msc_r[asl] = ms
                    upd = ms > bvs[p]
                    bvs[p] = jnp.where(upd, ms, bvs[p])
                    bps[p] = jnp.where(upd, gpos, bps[p])

            rec = jnp.zeros((L,), jnp.float32)
            for p in range(CL):
                rec = pack_rec(p, bvs[p], bps[p], rec)
            rec_r[...] = rec
            cp = pltpu.make_async_copy(
                rec_r.at[pl.ds(0, RW)],
                cand_sh.at[pl.ds(nslot + sid * RW, RW)], sem2)
            cp.start()

            @pl.when(sid == 0)
            def _():
                for p, c in enumerate(classes):
                    gmax, validb, wg, wg_i = winners[p][:4]
                    scorep = jnp.where(validb, gmax, 0.0)
                    lab = jnp.where(validb, jnp.float32(c), 0.0)
                    patch = jnp.where(
                        lanes == 7, scorep,
                        jnp.where(lanes == 8, lab, jnp.float32(0.0)))
                    patch_r[pl.ds(i * L + p * M * L, L)] = patch
                    vmask_r[pl.ds(i * L + p * M * L, L)] = jnp.where(
                        validb, 1.0, 0.0)
                    plsc.store_scatter(
                        idx_r, [jnp.broadcast_to(p * MPAD + i, (L,))],
                        wg_i, mask=lanes == 0)

            cp.wait()
            plsc.subcore_barrier()
            return carry

        lax.fori_loop(0, M, round_body, 0)

        @pl.when(sid == 0)
        def _():
            for p in range(CL):
                pltpu.async_copy(
                    bbp_hbm.at[idx_r.at[pl.ds(p * MPAD, M)]], rows_r,
                    sem).wait()

                def emit_body(r, carry3):
                    row16 = plsc.load_gather(
                        rows_r, [jnp.broadcast_to(r, (L,)), lanes])
                    psl = pl.ds(r * L + p * M * L, L)
                    outb_r[psl] = row16 * vmask_r[psl] + patch_r[psl]
                    return carry3

                lax.fori_loop(0, M, emit_body, 0)
            pltpu.sync_copy(outb_r.at[pl.ds(0, CL * M * L)],
                            out_hbm.at[pl.ds(out_off * L, CL * M * L)])

    @pl.when(cid == 0)
    def _():
        core_run((0, 1), 0)

    @pl.when(cid == 1)
    def _():
        core_run((2,), 2 * M)


@jax.jit
def kernel(mlvl_bboxes, mlvl_bboxes_for_nms, mlvl_scores):
    scT = jnp.zeros((C, NPAD), jnp.float32).at[:, :N].set(
        mlvl_scores[:, :C].T).reshape(C * NPAD)
    nmsT = jnp.zeros((4, NPAD), jnp.float32).at[:, :N].set(
        mlvl_bboxes_for_nms[:, :4].T).reshape(4 * NPAD)
    bbp = jnp.zeros((NPAD, 128), jnp.float32).at[:N, :7].set(mlvl_bboxes)
    mesh = plsc.VectorSubcoreMesh(core_axis_name="c", subcore_axis_name="s")
    out = pl.kernel(
        _sc_body,
        out_type=jax.ShapeDtypeStruct((C * M * L,), jnp.float32),
        mesh=mesh,
        compiler_params=pltpu.CompilerParams(needs_layout_passes=False),
        scratch_types=[
            pltpu.VMEM((NPAD,), jnp.float32),
            pltpu.VMEM((NPAD,), jnp.float32),
            pltpu.VMEM((NPAD,), jnp.float32),
            pltpu.VMEM((NPAD,), jnp.float32),
            pltpu.VMEM((PB,), jnp.float32),
            pltpu.VMEM((PB,), jnp.float32),
            pltpu.VMEM((PB,), jnp.float32),
            pltpu.VMEM((PB,), jnp.float32),
            pltpu.VMEM((PB,), jnp.float32),
            pltpu.VMEM((PB,), jnp.float32),
            pltpu.VMEM((C * PB,), jnp.float32),
            pltpu.VMEM((L,), jnp.float32),
            pltpu.VMEM_SHARED((2 * SLOT,), jnp.float32),
            pltpu.VMEM((SLOT,), jnp.float32),
            pltpu.VMEM((C * MPAD,), jnp.int32),
            pltpu.VMEM((C * M * L,), jnp.float32),
            pltpu.VMEM((C * M * L,), jnp.float32),
            pltpu.VMEM((M, 128), jnp.float32),
            pltpu.VMEM((C * M * L,), jnp.float32),
            pltpu.SemaphoreType.DMA,
            pltpu.SemaphoreType.DMA,
        ],
    )(bbp, nmsT, scT)
    return out.reshape(C * M, L)[:, :9]
